# baseline (speedup 1.0000x reference)
import jax
import jax.numpy as jnp
from jax.experimental import pallas as pl
from jax.experimental.pallas import tpu as pltpu

S = 2048
D = 2048
E = 16
CAP = 256
EC = E * CAP
LB_W = 0.01
EPS = 1.1920929e-07

_CHUNK = 256
_WBLK = 256
_KBLK = 256
_NK = D // _KBLK
_NW = S // (2 * _WBLK)


def _router(logits, aux_ref, scal_ref):
    iota_e = jax.lax.broadcasted_iota(jnp.int32, (S, E), 1)
    m = jnp.max(logits, axis=1, keepdims=True)
    ex = jnp.exp(logits - m)
    denom = jnp.sum(ex, axis=1, keepdims=True)
    gates = ex / denom

    mg = jnp.max(gates, axis=1, keepdims=True)
    e1 = jnp.min(jnp.where(gates == mg, iota_e, E), axis=1, keepdims=True)

    lm = jnp.where(iota_e == e1, -jnp.inf, logits)
    m2 = jnp.max(lm, axis=1, keepdims=True)
    e2 = jnp.min(jnp.where(lm == m2, iota_e, E), axis=1, keepdims=True)

    g2 = jnp.sum(jnp.where(iota_e == e2, gates, 0.0), axis=1, keepdims=True)

    oh1 = (iota_e == e1).astype(jnp.float32)
    count1_row = jnp.sum(oh1, axis=0, keepdims=True)
    me = jnp.sum(gates, axis=0, keepdims=True) * (1.0 / S)
    aux_ref[...] = ((E * LB_W / S) * jnp.sum(me * count1_row)).reshape(1, 1)

    e1f = e1.astype(jnp.float32)
    e2f = e2.astype(jnp.float32)
    mcol = jnp.concatenate(
        [mg, g2, e1f, e2f, jnp.zeros((S, 4), jnp.float32)], axis=1)
    mrow = mcol.T
    mg_row = mrow[0:1, :]
    g2_row = mrow[1:2, :]
    e1_row = mrow[2:3, :]
    e2_row = mrow[3:4, :]

    idx_col = jax.lax.broadcasted_iota(jnp.int32, (S, 1), 0)

    loc1_parts = []
    loc2_parts = []
    for ci in range(S // _CHUNK):
        a = ci * _CHUNK
        mg_i = mg_row[:, a:a + _CHUNK]
        e1_i = e1_row[:, a:a + _CHUNK]
        e2_i = e2_row[:, a:a + _CHUNK]
        idx_i = jax.lax.broadcasted_iota(jnp.int32, (1, _CHUNK), 1) + a

        before1 = (mg > mg_i) | ((mg == mg_i) & (idx_col < idx_i))
        hit1 = before1 & (e1f == e1_i)
        loc1_parts.append(
            jnp.sum(hit1.astype(jnp.float32), axis=0, keepdims=True))
        hit2 = (idx_col < idx_i) & (e2f == e2_i)
        loc2_parts.append(
            jnp.sum(hit2.astype(jnp.float32), axis=0, keepdims=True))
    loc1_row = jnp.concatenate(loc1_parts, axis=1)
    loc2_row = jnp.concatenate(loc2_parts, axis=1)

    iota_ec = jax.lax.broadcasted_iota(jnp.int32, (E, 1), 0).astype(jnp.float32)
    count1_col = jnp.sum((e1_row == iota_ec).astype(jnp.float32),
                         axis=1, keepdims=True)
    loc2_row = loc2_row + jnp.sum(
        jnp.where(e2_row == iota_ec, count1_col, 0.0),
        axis=0, keepdims=True)

    keep1 = (loc1_row < CAP).astype(jnp.float32)
    keep2 = (loc2_row < CAP).astype(jnp.float32)
    g1k = mg_row * keep1
    g2k = g2_row * keep2
    den2 = jnp.maximum(g1k + g2k, EPS)
    srow = jnp.concatenate(
        [e1_row, loc1_row * keep1, g1k / den2,
         e2_row, loc2_row * keep2, g2k / den2,
         jnp.zeros((2, S), jnp.float32)], axis=0)
    scal_ref[...] = srow.T


def _tile(scal_ref, blk):
    s = scal_ref[pl.ds(blk * _WBLK, _WBLK), :]
    e1 = s[:, 0:1].reshape(_WBLK, 1, 1)
    c1 = s[:, 1:2].reshape(_WBLK, 1, 1)
    v1 = s[:, 2:3].reshape(_WBLK, 1, 1)
    e2 = s[:, 3:4].reshape(_WBLK, 1, 1)
    c2 = s[:, 4:5].reshape(_WBLK, 1, 1)
    v2 = s[:, 5:6].reshape(_WBLK, 1, 1)
    eio = jax.lax.broadcasted_iota(jnp.int32, (_WBLK, E, 1), 1).astype(jnp.float32)
    cio = jax.lax.broadcasted_iota(jnp.int32, (_WBLK, 1, CAP), 2).astype(jnp.float32)
    a1 = jnp.where(eio == e1, v1, 0.0)
    a2 = jnp.where(eio == e2, v2, 0.0)
    b1 = (cio == c1).astype(jnp.float32)
    b2 = (cio == c2).astype(jnp.float32)
    comb = a1 * b1 + a2 * b2
    return comb, comb != 0.0


def _fused_body(x_ref, wt_ref, b_ref, comb_ref, disp_ref, aux_ref,
                logits_ref, scal_ref, cb0, cb1, sc0, sc1):
    i = pl.program_id(0)

    @pl.when(i == 0)
    def _():
        logits_ref[...] = jnp.dot(x_ref[...], wt_ref[...],
                                  preferred_element_type=jnp.float32)

    @pl.when((i > 0) & (i < _NK))
    def _():
        logits_ref[...] = logits_ref[...] + jnp.dot(
            x_ref[...], wt_ref[...], preferred_element_type=jnp.float32)

    @pl.when(i == _NK - 1)
    def _():
        _router(logits_ref[...] + b_ref[...], aux_ref, scal_ref)

    @pl.when(i >= _NK)
    def _():
        k = i - _NK
        b0 = 2 * k
        b1 = 2 * k + 1

        @pl.when(k > 0)
        def _():
            pltpu.make_async_copy(
                cb0, comb_ref.at[pl.ds((b0 - 2) * _WBLK, _WBLK)], sc0).wait()

        c0, d0 = _tile(scal_ref, b0)
        cb0[...] = c0
        disp_ref[0:_WBLK] = d0
        pltpu.make_async_copy(
            cb0, comb_ref.at[pl.ds(b0 * _WBLK, _WBLK)], sc0).start()

        @pl.when(k > 0)
        def _():
            pltpu.make_async_copy(
                cb1, comb_ref.at[pl.ds((b1 - 2) * _WBLK, _WBLK)], sc1).wait()

        c1, d1 = _tile(scal_ref, b1)
        cb1[...] = c1
        disp_ref[_WBLK:2 * _WBLK] = d1
        pltpu.make_async_copy(
            cb1, comb_ref.at[pl.ds(b1 * _WBLK, _WBLK)], sc1).start()

        @pl.when(k == _NW - 1)
        def _():
            pltpu.make_async_copy(
                cb0, comb_ref.at[pl.ds(b0 * _WBLK, _WBLK)], sc0).wait()
            pltpu.make_async_copy(
                cb1, comb_ref.at[pl.ds(b1 * _WBLK, _WBLK)], sc1).wait()


def kernel(x, W, b):
    wt = W.T
    b2 = b.reshape(1, E)
    comb, disp, aux = pl.pallas_call(
        _fused_body,
        grid=(_NK + _NW,),
        in_specs=[pl.BlockSpec((S, _KBLK),
                               lambda i: (0, jnp.minimum(i, _NK - 1))),
                  pl.BlockSpec((_KBLK, E),
                               lambda i: (jnp.minimum(i, _NK - 1), 0)),
                  pl.BlockSpec((1, E), lambda i: (0, 0))],
        out_specs=[pl.BlockSpec(memory_space=pl.ANY),
                   pl.BlockSpec((2 * _WBLK, E, CAP),
                                lambda i: (jnp.maximum(i - _NK, 0), 0, 0)),
                   pl.BlockSpec((1, 1), lambda i: (0, 0))],
        out_shape=[jax.ShapeDtypeStruct((S, E, CAP), jnp.float32),
                   jax.ShapeDtypeStruct((S, E, CAP), jnp.bool_),
                   jax.ShapeDtypeStruct((1, 1), jnp.float32)],
        scratch_shapes=[pltpu.VMEM((S, E), jnp.float32),
                        pltpu.VMEM((S, 8), jnp.float32),
                        pltpu.VMEM((_WBLK, E, CAP), jnp.float32),
                        pltpu.VMEM((_WBLK, E, CAP), jnp.float32),
                        pltpu.SemaphoreType.DMA,
                        pltpu.SemaphoreType.DMA],
    )(x, wt, b2)
    return aux[0, 0], comb, disp


# speedup vs baseline: 1.0097x; 1.0097x over previous
import jax
import jax.numpy as jnp
from jax.experimental import pallas as pl
from jax.experimental.pallas import tpu as pltpu

S = 2048
D = 2048
E = 16
CAP = 256
EC = E * CAP
LB_W = 0.01
EPS = 1.1920929e-07

_CHUNK = 256
_WBLK = 128
_KBLK = 256
_NK = D // _KBLK
_NW = S // (2 * _WBLK)


def _router(logits, aux_ref, scal_ref):
    iota_e = jax.lax.broadcasted_iota(jnp.int32, (S, E), 1)
    m = jnp.max(logits, axis=1, keepdims=True)
    ex = jnp.exp(logits - m)
    denom = jnp.sum(ex, axis=1, keepdims=True)
    gates = ex / denom

    mg = jnp.max(gates, axis=1, keepdims=True)
    e1 = jnp.min(jnp.where(gates == mg, iota_e, E), axis=1, keepdims=True)

    lm = jnp.where(iota_e == e1, -jnp.inf, logits)
    m2 = jnp.max(lm, axis=1, keepdims=True)
    e2 = jnp.min(jnp.where(lm == m2, iota_e, E), axis=1, keepdims=True)

    g2 = jnp.sum(jnp.where(iota_e == e2, gates, 0.0), axis=1, keepdims=True)

    oh1 = (iota_e == e1).astype(jnp.float32)
    count1_row = jnp.sum(oh1, axis=0, keepdims=True)
    me = jnp.sum(gates, axis=0, keepdims=True) * (1.0 / S)
    aux_ref[...] = ((E * LB_W / S) * jnp.sum(me * count1_row)).reshape(1, 1)

    e1f = e1.astype(jnp.float32)
    e2f = e2.astype(jnp.float32)
    mcol = jnp.concatenate(
        [mg, g2, e1f, e2f, jnp.zeros((S, 4), jnp.float32)], axis=1)
    mrow = mcol.T
    mg_row = mrow[0:1, :]
    g2_row = mrow[1:2, :]
    e1_row = mrow[2:3, :]
    e2_row = mrow[3:4, :]

    idx_col = jax.lax.broadcasted_iota(jnp.int32, (S, 1), 0)

    loc1_parts = []
    loc2_parts = []
    for ci in range(S // _CHUNK):
        a = ci * _CHUNK
        mg_i = mg_row[:, a:a + _CHUNK]
        e1_i = e1_row[:, a:a + _CHUNK]
        e2_i = e2_row[:, a:a + _CHUNK]
        idx_i = jax.lax.broadcasted_iota(jnp.int32, (1, _CHUNK), 1) + a

        before1 = (mg > mg_i) | ((mg == mg_i) & (idx_col < idx_i))
        hit1 = before1 & (e1f == e1_i)
        loc1_parts.append(
            jnp.sum(hit1.astype(jnp.float32), axis=0, keepdims=True))
        hit2 = (idx_col < idx_i) & (e2f == e2_i)
        loc2_parts.append(
            jnp.sum(hit2.astype(jnp.float32), axis=0, keepdims=True))
    loc1_row = jnp.concatenate(loc1_parts, axis=1)
    loc2_row = jnp.concatenate(loc2_parts, axis=1)

    iota_ec = jax.lax.broadcasted_iota(jnp.int32, (E, 1), 0).astype(jnp.float32)
    count1_col = jnp.sum((e1_row == iota_ec).astype(jnp.float32),
                         axis=1, keepdims=True)
    loc2_row = loc2_row + jnp.sum(
        jnp.where(e2_row == iota_ec, count1_col, 0.0),
        axis=0, keepdims=True)

    keep1 = (loc1_row < CAP).astype(jnp.float32)
    keep2 = (loc2_row < CAP).astype(jnp.float32)
    g1k = mg_row * keep1
    g2k = g2_row * keep2
    den2 = jnp.maximum(g1k + g2k, EPS)
    srow = jnp.concatenate(
        [e1_row, loc1_row * keep1, g1k / den2,
         e2_row, loc2_row * keep2, g2k / den2,
         jnp.zeros((2, S), jnp.float32)], axis=0)
    scal_ref[...] = srow.T


def _tile(scal_ref, blk):
    s = scal_ref[pl.ds(blk * _WBLK, _WBLK), :]
    e1 = s[:, 0:1].reshape(_WBLK, 1, 1)
    c1 = s[:, 1:2].reshape(_WBLK, 1, 1)
    v1 = s[:, 2:3].reshape(_WBLK, 1, 1)
    e2 = s[:, 3:4].reshape(_WBLK, 1, 1)
    c2 = s[:, 4:5].reshape(_WBLK, 1, 1)
    v2 = s[:, 5:6].reshape(_WBLK, 1, 1)
    eio = jax.lax.broadcasted_iota(jnp.int32, (_WBLK, E, 1), 1).astype(jnp.float32)
    cio = jax.lax.broadcasted_iota(jnp.int32, (_WBLK, 1, CAP), 2).astype(jnp.float32)
    a1 = jnp.where(eio == e1, v1, 0.0)
    a2 = jnp.where(eio == e2, v2, 0.0)
    b1 = (cio == c1).astype(jnp.float32)
    b2 = (cio == c2).astype(jnp.float32)
    comb = a1 * b1 + a2 * b2
    return comb, comb != 0.0


def _fused_body(x_ref, wt_ref, b_ref, comb_ref, disp_ref, aux_ref,
                logits_ref, scal_ref, cb0, cb1, sc0, sc1):
    i = pl.program_id(0)

    @pl.when(i == 0)
    def _():
        logits_ref[...] = jnp.dot(x_ref[...], wt_ref[...],
                                  preferred_element_type=jnp.float32)

    @pl.when((i > 0) & (i < _NK))
    def _():
        logits_ref[...] = logits_ref[...] + jnp.dot(
            x_ref[...], wt_ref[...], preferred_element_type=jnp.float32)

    @pl.when(i == _NK - 1)
    def _():
        _router(logits_ref[...] + b_ref[...], aux_ref, scal_ref)

    @pl.when(i >= _NK)
    def _():
        k = i - _NK
        b0 = 2 * k
        b1 = 2 * k + 1

        @pl.when(k > 0)
        def _():
            pltpu.make_async_copy(
                cb0, comb_ref.at[pl.ds((b0 - 2) * _WBLK, _WBLK)], sc0).wait()

        c0, d0 = _tile(scal_ref, b0)
        cb0[...] = c0
        disp_ref[0:_WBLK] = d0
        pltpu.make_async_copy(
            cb0, comb_ref.at[pl.ds(b0 * _WBLK, _WBLK)], sc0).start()

        @pl.when(k > 0)
        def _():
            pltpu.make_async_copy(
                cb1, comb_ref.at[pl.ds((b1 - 2) * _WBLK, _WBLK)], sc1).wait()

        c1, d1 = _tile(scal_ref, b1)
        cb1[...] = c1
        disp_ref[_WBLK:2 * _WBLK] = d1
        pltpu.make_async_copy(
            cb1, comb_ref.at[pl.ds(b1 * _WBLK, _WBLK)], sc1).start()

        @pl.when(k == _NW - 1)
        def _():
            pltpu.make_async_copy(
                cb0, comb_ref.at[pl.ds(b0 * _WBLK, _WBLK)], sc0).wait()
            pltpu.make_async_copy(
                cb1, comb_ref.at[pl.ds(b1 * _WBLK, _WBLK)], sc1).wait()


def kernel(x, W, b):
    wt = W.T
    b2 = b.reshape(1, E)
    comb, disp, aux = pl.pallas_call(
        _fused_body,
        grid=(_NK + _NW,),
        in_specs=[pl.BlockSpec((S, _KBLK),
                               lambda i: (0, jnp.minimum(i, _NK - 1))),
                  pl.BlockSpec((_KBLK, E),
                               lambda i: (jnp.minimum(i, _NK - 1), 0)),
                  pl.BlockSpec((1, E), lambda i: (0, 0))],
        out_specs=[pl.BlockSpec(memory_space=pl.ANY),
                   pl.BlockSpec((2 * _WBLK, E, CAP),
                                lambda i: (jnp.maximum(i - _NK, 0), 0, 0)),
                   pl.BlockSpec((1, 1), lambda i: (0, 0))],
        out_shape=[jax.ShapeDtypeStruct((S, E, CAP), jnp.float32),
                   jax.ShapeDtypeStruct((S, E, CAP), jnp.bool_),
                   jax.ShapeDtypeStruct((1, 1), jnp.float32)],
        scratch_shapes=[pltpu.VMEM((S, E), jnp.float32),
                        pltpu.VMEM((S, 8), jnp.float32),
                        pltpu.VMEM((_WBLK, E, CAP), jnp.float32),
                        pltpu.VMEM((_WBLK, E, CAP), jnp.float32),
                        pltpu.SemaphoreType.DMA,
                        pltpu.SemaphoreType.DMA],
    )(x, wt, b2)
    return aux[0, 0], comb, disp


# speedup vs baseline: 1.1394x; 1.1284x over previous
import jax
import jax.numpy as jnp
from jax.experimental import pallas as pl
from jax.experimental.pallas import tpu as pltpu

S = 2048
D = 2048
E = 16
CAP = 256
EC = E * CAP
LB_W = 0.01
EPS = 1.1920929e-07

_CHUNK = 256
_WBLK = 128
_KBLK = 256
_NK = D // _KBLK
_NW = S // (2 * _WBLK)


def _router(logits, aux_ref, scal_ref):
    iota_e = jax.lax.broadcasted_iota(jnp.int32, (S, E), 1)
    m = jnp.max(logits, axis=1, keepdims=True)
    ex = jnp.exp(logits - m)
    denom = jnp.sum(ex, axis=1, keepdims=True)
    gates = ex / denom

    mg = jnp.max(gates, axis=1, keepdims=True)
    e1 = jnp.min(jnp.where(gates == mg, iota_e, E), axis=1, keepdims=True)

    lm = jnp.where(iota_e == e1, -jnp.inf, logits)
    m2 = jnp.max(lm, axis=1, keepdims=True)
    e2 = jnp.min(jnp.where(lm == m2, iota_e, E), axis=1, keepdims=True)

    g2 = jnp.sum(jnp.where(iota_e == e2, gates, 0.0), axis=1, keepdims=True)

    oh1 = (iota_e == e1).astype(jnp.float32)
    count1_row = jnp.sum(oh1, axis=0, keepdims=True)
    me = jnp.sum(gates, axis=0, keepdims=True) * (1.0 / S)
    aux_ref[...] = ((E * LB_W / S) * jnp.sum(me * count1_row)).reshape(1, 1)

    e1f = e1.astype(jnp.float32)
    e2f = e2.astype(jnp.float32)
    mcol = jnp.concatenate(
        [mg, g2, e1f, e2f, jnp.zeros((S, 4), jnp.float32)], axis=1)
    mrow = mcol.T
    mg_row = mrow[0:1, :]
    g2_row = mrow[1:2, :]
    e1_row = mrow[2:3, :]
    e2_row = mrow[3:4, :]

    idx_col = jax.lax.broadcasted_iota(jnp.int32, (S, 1), 0)

    loc1_parts = []
    loc2_parts = []
    for ci in range(S // _CHUNK):
        a = ci * _CHUNK
        mg_i = mg_row[:, a:a + _CHUNK]
        e1_i = e1_row[:, a:a + _CHUNK]
        e2_i = e2_row[:, a:a + _CHUNK]
        idx_i = jax.lax.broadcasted_iota(jnp.int32, (1, _CHUNK), 1) + a

        before1 = (mg > mg_i) | ((mg == mg_i) & (idx_col < idx_i))
        hit1 = before1 & (e1f == e1_i)
        loc1_parts.append(
            jnp.sum(hit1.astype(jnp.float32), axis=0, keepdims=True))
        hit2 = (idx_col < idx_i) & (e2f == e2_i)
        loc2_parts.append(
            jnp.sum(hit2.astype(jnp.float32), axis=0, keepdims=True))
    loc1_row = jnp.concatenate(loc1_parts, axis=1)
    loc2_row = jnp.concatenate(loc2_parts, axis=1)

    iota_ec = jax.lax.broadcasted_iota(jnp.int32, (E, 1), 0).astype(jnp.float32)
    count1_col = jnp.sum((e1_row == iota_ec).astype(jnp.float32),
                         axis=1, keepdims=True)
    loc2_row = loc2_row + jnp.sum(
        jnp.where(e2_row == iota_ec, count1_col, 0.0),
        axis=0, keepdims=True)

    keep1 = (loc1_row < CAP).astype(jnp.float32)
    keep2 = (loc2_row < CAP).astype(jnp.float32)
    g1k = mg_row * keep1
    g2k = g2_row * keep2
    den2 = jnp.maximum(g1k + g2k, EPS)
    srow = jnp.concatenate(
        [e1_row, loc1_row * keep1, g1k / den2,
         e2_row, loc2_row * keep2, g2k / den2,
         jnp.zeros((2, S), jnp.float32)], axis=0)
    scal_ref[...] = srow.T


def _tile(scal_ref, blk):
    s = scal_ref[pl.ds(blk * _WBLK, _WBLK), :]
    e1 = s[:, 0:1].reshape(_WBLK, 1, 1)
    c1 = s[:, 1:2].reshape(_WBLK, 1, 1)
    v1 = s[:, 2:3].reshape(_WBLK, 1, 1)
    e2 = s[:, 3:4].reshape(_WBLK, 1, 1)
    c2 = s[:, 4:5].reshape(_WBLK, 1, 1)
    v2 = s[:, 5:6].reshape(_WBLK, 1, 1)
    eio = jax.lax.broadcasted_iota(jnp.int32, (_WBLK, E, 1), 1).astype(jnp.float32)
    cio = jax.lax.broadcasted_iota(jnp.int32, (_WBLK, 1, CAP), 2).astype(jnp.float32)
    a1 = jnp.where(eio == e1, v1, 0.0)
    a2 = jnp.where(eio == e2, v2, 0.0)
    b1 = (cio == c1).astype(jnp.float32)
    b2 = (cio == c2).astype(jnp.float32)
    return a1 * b1 + a2 * b2


def _fused_body(x_ref, w_ref, b_ref, comb_ref, aux_ref,
                logits_ref, scal_ref, cb0, cb1, sc0, sc1):
    i = pl.program_id(0)

    dn = (((1,), (1,)), ((), ()))

    @pl.when(i == 0)
    def _():
        logits_ref[...] = jax.lax.dot_general(
            x_ref[...], w_ref[...], dn, preferred_element_type=jnp.float32)

    @pl.when((i > 0) & (i < _NK))
    def _():
        logits_ref[...] = logits_ref[...] + jax.lax.dot_general(
            x_ref[...], w_ref[...], dn, preferred_element_type=jnp.float32)

    @pl.when(i == _NK - 1)
    def _():
        _router(logits_ref[...] + b_ref[...], aux_ref, scal_ref)

    @pl.when(i >= _NK)
    def _():
        k = i - _NK
        b0 = 2 * k
        b1 = 2 * k + 1

        @pl.when(k > 0)
        def _():
            pltpu.make_async_copy(
                cb0, comb_ref.at[pl.ds((b0 - 2) * _WBLK, _WBLK)], sc0).wait()

        c0 = _tile(scal_ref, b0)
        cb0[...] = c0
        pltpu.make_async_copy(
            cb0, comb_ref.at[pl.ds(b0 * _WBLK, _WBLK)], sc0).start()

        @pl.when(k > 0)
        def _():
            pltpu.make_async_copy(
                cb1, comb_ref.at[pl.ds((b1 - 2) * _WBLK, _WBLK)], sc1).wait()

        c1 = _tile(scal_ref, b1)
        cb1[...] = c1
        pltpu.make_async_copy(
            cb1, comb_ref.at[pl.ds(b1 * _WBLK, _WBLK)], sc1).start()

        @pl.when(k == _NW - 1)
        def _():
            pltpu.make_async_copy(
                cb0, comb_ref.at[pl.ds(b0 * _WBLK, _WBLK)], sc0).wait()
            pltpu.make_async_copy(
                cb1, comb_ref.at[pl.ds(b1 * _WBLK, _WBLK)], sc1).wait()


def kernel(x, W, b):
    b2 = b.reshape(1, E)
    comb, aux = pl.pallas_call(
        _fused_body,
        grid=(_NK + _NW,),
        in_specs=[pl.BlockSpec((S, _KBLK),
                               lambda i: (0, jnp.minimum(i, _NK - 1))),
                  pl.BlockSpec((E, _KBLK),
                               lambda i: (0, jnp.minimum(i, _NK - 1))),
                  pl.BlockSpec((1, E), lambda i: (0, 0))],
        out_specs=[pl.BlockSpec(memory_space=pl.ANY),
                   pl.BlockSpec((1, 1), lambda i: (0, 0))],
        out_shape=[jax.ShapeDtypeStruct((S, E, CAP), jnp.float32),
                   jax.ShapeDtypeStruct((1, 1), jnp.float32)],
        scratch_shapes=[pltpu.VMEM((S, E), jnp.float32),
                        pltpu.VMEM((S, 8), jnp.float32),
                        pltpu.VMEM((_WBLK, E, CAP), jnp.float32),
                        pltpu.VMEM((_WBLK, E, CAP), jnp.float32),
                        pltpu.SemaphoreType.DMA,
                        pltpu.SemaphoreType.DMA],
    )(x, W, b2)
    return aux[0, 0], comb, comb.astype(jnp.bool_)
